# Initial kernel scaffold; baseline (speedup 1.0000x reference)
import jax
import jax.numpy as jnp
from jax.experimental import pallas as pl


def kernel(x, level1_labels, emb, W1, b1, W2, b2):
    raise NotImplementedError("write your pallas kernel here")


# speedup vs baseline: 2.4251x; 2.4251x over previous
import functools

import jax
import jax.numpy as jnp
from jax import lax
from jax.experimental import pallas as pl
from jax.experimental.pallas import tpu as pltpu
from jax.experimental.pallas import tpu_sc as plsc

VOCAB = 1000000
EMB = 64
NUM_L1 = 32
NUM_LEAF = 1024
B = 16384
L = 200
HALF = L // 2

NC, NS = 2, 16
NW = NC * NS
ROWS_PER_W = B // NW
PB = 4
P = 2 * PB
N_ITER = ROWS_PER_W // PB
LANES = 16
C_CHUNKS = EMB // LANES


def _gather_sum_kernel(x2_hbm, emb_hbm, out_hbm, idx_v, rows_v, pooled_v, sem):
    wid = lax.axis_index("s") * NC + lax.axis_index("c")
    base_half = wid * ROWS_PER_W * 2

    def iteration(i, _):
        pltpu.sync_copy(x2_hbm.at[pl.ds(base_half + i * P, P)], idx_v)
        descs = [
            pltpu.async_copy(
                emb_hbm.at[idx_v.at[j]],
                rows_v.at[pl.ds(j * HALF, HALF)],
                sem,
            )
            for j in range(P)
        ]
        for d in descs:
            d.wait()
        zero = jnp.zeros((LANES,), jnp.float32)
        for k in range(PB):
            rbase = k * L

            def body(r, acc, rbase=rbase):
                a = list(acc)
                for u in range(4):
                    rr = rbase + r * 4 + u
                    for c in range(C_CHUNKS):
                        a[c] = a[c] + rows_v[rr, pl.ds(c * LANES, LANES)]
                return tuple(a)

            acc = lax.fori_loop(0, L // 4, body, (zero,) * C_CHUNKS)
            for c in range(C_CHUNKS):
                pooled_v[i * PB + k, pl.ds(c * LANES, LANES)] = acc[c]
        return 0

    lax.fori_loop(0, N_ITER, iteration, 0)
    pltpu.sync_copy(pooled_v, out_hbm.at[pl.ds(wid * ROWS_PER_W, ROWS_PER_W)])


_gather_sum = functools.partial(
    pl.kernel,
    out_type=jax.ShapeDtypeStruct((B, EMB), jnp.float32),
    mesh=plsc.VectorSubcoreMesh(core_axis_name="c", subcore_axis_name="s"),
    scratch_types=[
        pltpu.VMEM((P, HALF), jnp.int32),
        pltpu.VMEM((P * HALF, EMB), jnp.float32),
        pltpu.VMEM((ROWS_PER_W, EMB), jnp.float32),
        pltpu.SemaphoreType.DMA,
    ],
    compiler_params=pltpu.CompilerParams(use_tc_tiling_on_sc=False),
)(_gather_sum_kernel)


BLK = 2048


def _dense_body(pooled_ref, x_ref, lab_ref, w1_ref, b1_ref, w2_ref, b2_ref,
                l1_ref, leaf_ref):
    cnt = jnp.sum((x_ref[...] != 0).astype(jnp.float32), axis=1, keepdims=True)
    h = jnp.maximum(pooled_ref[...] / cnt, 0.0)
    l1_ref[...] = (
        jnp.dot(h, w1_ref[...], preferred_element_type=jnp.float32)
        + b1_ref[...]
    )
    one_hot = (
        lab_ref[...]
        == lax.broadcasted_iota(jnp.int32, (BLK, NUM_L1), 1)
    ).astype(jnp.float32)
    leaf_ref[...] = (
        jnp.dot(h, w2_ref[0:EMB, :], preferred_element_type=jnp.float32)
        + jnp.dot(one_hot, w2_ref[EMB:, :], preferred_element_type=jnp.float32)
        + b2_ref[...]
    )


def kernel(x, level1_labels, emb, W1, b1, W2, b2):
    x2 = x.reshape(B * 2, HALF)
    pooled_sum = _gather_sum(x2, emb)

    lab2d = level1_labels.reshape(B, 1)
    grid = B // BLK
    l1, leaf = pl.pallas_call(
        _dense_body,
        grid=(grid,),
        in_specs=[
            pl.BlockSpec((BLK, EMB), lambda i: (i, 0)),
            pl.BlockSpec((BLK, L), lambda i: (i, 0)),
            pl.BlockSpec((BLK, 1), lambda i: (i, 0)),
            pl.BlockSpec((EMB, NUM_L1), lambda i: (0, 0)),
            pl.BlockSpec((1, NUM_L1), lambda i: (0, 0)),
            pl.BlockSpec((EMB + NUM_L1, NUM_LEAF), lambda i: (0, 0)),
            pl.BlockSpec((1, NUM_LEAF), lambda i: (0, 0)),
        ],
        out_specs=[
            pl.BlockSpec((BLK, NUM_L1), lambda i: (i, 0)),
            pl.BlockSpec((BLK, NUM_LEAF), lambda i: (i, 0)),
        ],
        out_shape=[
            jax.ShapeDtypeStruct((B, NUM_L1), jnp.float32),
            jax.ShapeDtypeStruct((B, NUM_LEAF), jnp.float32),
        ],
    )(pooled_sum, x, lab2d, W1, b1.reshape(1, NUM_L1), W2,
      b2.reshape(1, NUM_LEAF))
    return (l1, leaf)


# speedup vs baseline: 3.1605x; 1.3033x over previous
import functools

import jax
import jax.numpy as jnp
from jax import lax
from jax.experimental import pallas as pl
from jax.experimental.pallas import tpu as pltpu
from jax.experimental.pallas import tpu_sc as plsc

VOCAB = 1000000
EMB = 64
NUM_L1 = 32
NUM_LEAF = 1024
B = 16384
L = 200
SPLIT = 104

NC, NS = 2, 16
NW = NC * NS
ROWS_PER_W = B // NW
PB = 2
N_STEP = ROWS_PER_W // PB
LANES = 16
C_CHUNKS = EMB // LANES
UNROLL = 8


def _gather_sum_kernel(x_hbm, emb_hbm, out_hbm,
                       idx0, idx1, rows0, rows1, pooled_v,
                       isem0, isem1, gsem0, gsem1):
    idx_v = (idx0, idx1)
    rows_v = (rows0, rows1)
    isem = (isem0, isem1)
    gsem = (gsem0, gsem1)
    wid = lax.axis_index("s") * NC + lax.axis_index("c")
    base = wid * ROWS_PER_W

    def idx_start(slot, step):
        pltpu.async_copy(
            x_hbm.at[pl.ds(base + step * PB, PB)], idx_v[slot], isem[slot])

    def idx_wait(slot):
        pltpu.make_async_copy(
            x_hbm.at[pl.ds(0, PB)], idx_v[slot], isem[slot]).wait()

    def gathers_start(slot):
        for k in range(PB):
            pltpu.async_copy(
                emb_hbm.at[idx_v[slot].at[k, pl.ds(0, SPLIT)]],
                rows_v[slot].at[pl.ds(k * L, SPLIT)],
                gsem[slot])
            pltpu.async_copy(
                emb_hbm.at[idx_v[slot].at[k, pl.ds(SPLIT, L - SPLIT)]],
                rows_v[slot].at[pl.ds(k * L + SPLIT, L - SPLIT)],
                gsem[slot])

    def gathers_wait(slot):
        pltpu.make_async_copy(
            emb_hbm.at[pl.ds(0, PB * L)], rows_v[slot], gsem[slot]).wait()

    def reduce(slot, step):
        zero = jnp.zeros((LANES,), jnp.float32)
        for k in range(PB):
            rbase = k * L

            def body(r, acc, rbase=rbase, slot=slot):
                a = list(acc)
                for u in range(UNROLL):
                    rr = rbase + r * UNROLL + u
                    for c in range(C_CHUNKS):
                        a[c] = a[c] + rows_v[slot][rr, pl.ds(c * LANES, LANES)]
                return tuple(a)

            acc = lax.fori_loop(0, L // UNROLL, body, (zero,) * C_CHUNKS)
            for c in range(C_CHUNKS):
                pooled_v[step * PB + k, pl.ds(c * LANES, LANES)] = acc[c]

    idx_start(0, 0)
    idx_wait(0)
    gathers_start(0)
    idx_start(1, 1)

    def body(i2, _):
        s0 = 2 * i2
        s1 = s0 + 1
        idx_wait(1)
        gathers_start(1)
        gathers_wait(0)

        @pl.when(s0 + 2 < N_STEP)
        def _():
            idx_start(0, s0 + 2)

        reduce(0, s0)

        @pl.when(s0 + 2 < N_STEP)
        def _():
            idx_wait(0)
            gathers_start(0)

        gathers_wait(1)

        @pl.when(s1 + 2 < N_STEP)
        def _():
            idx_start(1, s1 + 2)

        reduce(1, s1)
        return 0

    lax.fori_loop(0, N_STEP // 2, body, 0)
    pltpu.sync_copy(pooled_v, out_hbm.at[pl.ds(base, ROWS_PER_W)])


_gather_sum = functools.partial(
    pl.kernel,
    out_type=jax.ShapeDtypeStruct((B, EMB), jnp.float32),
    mesh=plsc.VectorSubcoreMesh(core_axis_name="c", subcore_axis_name="s"),
    scratch_types=[
        pltpu.VMEM((PB, L), jnp.int32),
        pltpu.VMEM((PB, L), jnp.int32),
        pltpu.VMEM((PB * L, EMB), jnp.float32),
        pltpu.VMEM((PB * L, EMB), jnp.float32),
        pltpu.VMEM((ROWS_PER_W, EMB), jnp.float32),
        pltpu.SemaphoreType.DMA,
        pltpu.SemaphoreType.DMA,
        pltpu.SemaphoreType.DMA,
        pltpu.SemaphoreType.DMA,
    ],
    compiler_params=pltpu.CompilerParams(use_tc_tiling_on_sc=False),
)(_gather_sum_kernel)


BLK = 2048


def _dense_body(pooled_ref, x_ref, lab_ref, w1_ref, b1_ref, w2_ref, b2_ref,
                l1_ref, leaf_ref):
    cnt = jnp.sum((x_ref[...] != 0).astype(jnp.float32), axis=1, keepdims=True)
    h = jnp.maximum(pooled_ref[...] / cnt, 0.0)
    l1_ref[...] = (
        jnp.dot(h, w1_ref[...], preferred_element_type=jnp.float32)
        + b1_ref[...]
    )
    one_hot = (
        lab_ref[...]
        == lax.broadcasted_iota(jnp.int32, (BLK, NUM_L1), 1)
    ).astype(jnp.float32)
    leaf_ref[...] = (
        jnp.dot(h, w2_ref[0:EMB, :], preferred_element_type=jnp.float32)
        + jnp.dot(one_hot, w2_ref[EMB:, :], preferred_element_type=jnp.float32)
        + b2_ref[...]
    )


def kernel(x, level1_labels, emb, W1, b1, W2, b2):
    pooled_sum = _gather_sum(x, emb)

    lab2d = level1_labels.reshape(B, 1)
    grid = B // BLK
    l1, leaf = pl.pallas_call(
        _dense_body,
        grid=(grid,),
        in_specs=[
            pl.BlockSpec((BLK, EMB), lambda i: (i, 0)),
            pl.BlockSpec((BLK, L), lambda i: (i, 0)),
            pl.BlockSpec((BLK, 1), lambda i: (i, 0)),
            pl.BlockSpec((EMB, NUM_L1), lambda i: (0, 0)),
            pl.BlockSpec((1, NUM_L1), lambda i: (0, 0)),
            pl.BlockSpec((EMB + NUM_L1, NUM_LEAF), lambda i: (0, 0)),
            pl.BlockSpec((1, NUM_LEAF), lambda i: (0, 0)),
        ],
        out_specs=[
            pl.BlockSpec((BLK, NUM_L1), lambda i: (i, 0)),
            pl.BlockSpec((BLK, NUM_LEAF), lambda i: (i, 0)),
        ],
        out_shape=[
            jax.ShapeDtypeStruct((B, NUM_L1), jnp.float32),
            jax.ShapeDtypeStruct((B, NUM_LEAF), jnp.float32),
        ],
    )(pooled_sum, x, lab2d, W1, b1.reshape(1, NUM_L1), W2,
      b2.reshape(1, NUM_LEAF))
    return (l1, leaf)
